# baseline (speedup 1.0000x reference)
import functools

import jax
import jax.numpy as jnp
from jax import lax
from jax.experimental import pallas as pl
from jax.experimental.pallas import tpu as pltpu
from jax.experimental.pallas import tpu_sc as plsc

_K = 20
_RB = 256


def _topk_kernel(x_ref, idx_ref):
    nb = pl.program_id(1)
    n0 = nb * _RB
    _, d, N = x_ref.shape

    x8 = x_ref[0, 0:8, :]
    xr8 = x_ref[0, 0:8, pl.ds(n0, _RB)]
    row = lax.broadcasted_iota(jnp.int32, (8, N), 0)
    rowr = lax.broadcasted_iota(jnp.int32, (8, _RB), 0)

    x3 = jnp.where(row < 3, x8, 0.0)
    xr3 = jnp.where(rowr < 3, 2.0 * xr8, 0.0)

    inner2 = lax.dot_general(x3, xr3, (((0,), (0,)), ((), ())),
                             preferred_element_type=jnp.float32)
    ones_col = jnp.ones((8, 1), jnp.float32)
    xxcol = lax.dot_general(x3 * x3, ones_col, (((0,), (0,)), ((), ())),
                            precision=lax.Precision.HIGHEST,
                            preferred_element_type=jnp.float32)
    p = inner2 - xxcol

    sub = lax.broadcasted_iota(jnp.int32, (N, _RB), 0)
    neg = jnp.float32(-jnp.inf)
    for t in range(_K):
        it = jnp.argmax(p, axis=0).astype(jnp.int32)[None, :]
        idx_ref[0, t, :] = it[0]
        p = jnp.where(sub == it, neg, p)


def _topk(x):
    B, d, N = x.shape
    return pl.pallas_call(
        _topk_kernel,
        grid=(B, N // _RB),
        in_specs=[pl.BlockSpec((1, d, N), lambda b, nb: (b, 0, 0))],
        out_specs=pl.BlockSpec((1, _K, _RB), lambda b, nb: (b, 0, nb)),
        out_shape=jax.ShapeDtypeStruct((B, _K, N), jnp.int32),
    )(x)


def _make_sc_gather(B, d, N):
    KN = _K * N
    n_groups = N // 16
    mesh = plsc.VectorSubcoreMesh(core_axis_name="c", subcore_axis_name="s")

    @functools.partial(
        pl.kernel,
        mesh=mesh,
        compiler_params=pltpu.CompilerParams(needs_layout_passes=False),
        out_type=jax.ShapeDtypeStruct((B * 2 * d * KN,), jnp.float32),
        scratch_types=[
            pltpu.VMEM((KN,), jnp.int32),
            pltpu.VMEM((N,), jnp.float32),
            pltpu.VMEM((KN,), jnp.float32),
        ],
    )
    def sc_gather(xf_hbm, idxf_hbm, outf_hbm, idxbuf, xrow, rowbuf):
        wid = lax.axis_index("s") * 2 + lax.axis_index("c")
        b = wid // 4
        cg = wid % 4
        pltpu.sync_copy(idxf_hbm.at[pl.ds(b * KN, KN)], idxbuf)

        def gather_group(g, t):
            iv = idxbuf[pl.ds(t * N + g * 16, 16)]
            fv = plsc.load_gather(xrow, [iv])
            cv = xrow[pl.ds(g * 16, 16)]
            rowbuf[pl.ds(t * N + g * 16, 16)] = fv - cv
            return t

        def copy_group(g, carry):
            v = xrow[pl.ds((g % n_groups) * 16, 16)]
            rowbuf[pl.ds(g * 16, 16)] = v
            return carry

        for ci in range(16):
            c = 16 * cg + ci
            pltpu.sync_copy(xf_hbm.at[pl.ds((b * d + c) * N, N)], xrow)
            def t_body(t, carry):
                lax.fori_loop(0, n_groups, gather_group, t, unroll=8)
                return carry
            lax.fori_loop(0, _K, t_body, 0)
            pltpu.sync_copy(rowbuf,
                            outf_hbm.at[pl.ds(((b * 2 * d) + c) * KN, KN)])
            lax.fori_loop(0, _K * n_groups, copy_group, 0, unroll=8)
            pltpu.sync_copy(rowbuf,
                            outf_hbm.at[pl.ds(((b * 2 * d) + d + c) * KN, KN)])

    return sc_gather


def kernel(x):
    B, d, N = x.shape
    idx = _topk(x)
    xf = x.reshape(-1)
    idxf = idx.reshape(-1)
    outf = _make_sc_gather(B, d, N)(xf, idxf)
    out = outf.reshape(B, 2 * d, _K, N)
    return jnp.transpose(out, (0, 1, 3, 2))


# speedup vs baseline: 1.0352x; 1.0352x over previous
import functools

import jax
import jax.numpy as jnp
from jax import lax
from jax.experimental import pallas as pl
from jax.experimental.pallas import tpu as pltpu
from jax.experimental.pallas import tpu_sc as plsc

_K = 20
_RB = 256


def _topk_kernel(x_ref, idx_ref):
    nb = pl.program_id(1)
    n0 = nb * _RB
    _, d, N = x_ref.shape

    x8 = x_ref[0, 0:8, :]
    xr8 = x_ref[0, 0:8, pl.ds(n0, _RB)]
    row = lax.broadcasted_iota(jnp.int32, (8, N), 0)
    rowr = lax.broadcasted_iota(jnp.int32, (8, _RB), 0)

    x3 = jnp.where(row < 3, x8, 0.0)
    xr3 = jnp.where(rowr < 3, 2.0 * xr8, 0.0)

    inner2 = lax.dot_general(x3, xr3, (((0,), (0,)), ((), ())),
                             preferred_element_type=jnp.float32)
    ones_col = jnp.ones((8, 1), jnp.float32)
    xxcol = lax.dot_general(x3 * x3, ones_col, (((0,), (0,)), ((), ())),
                            precision=lax.Precision.HIGHEST,
                            preferred_element_type=jnp.float32)
    p = inner2 - xxcol

    sub = lax.broadcasted_iota(jnp.int32, (N, _RB), 0)
    neg = jnp.float32(-jnp.inf)
    for t in range(_K):
        it = jnp.argmax(p, axis=0).astype(jnp.int32)[None, :]
        idx_ref[0, t, :] = it[0]
        p = jnp.where(sub == it, neg, p)


def _topk(x):
    B, d, N = x.shape
    return pl.pallas_call(
        _topk_kernel,
        grid=(B, N // _RB),
        in_specs=[pl.BlockSpec((1, d, N), lambda b, nb: (b, 0, 0))],
        out_specs=pl.BlockSpec((1, _K, _RB), lambda b, nb: (b, 0, nb)),
        out_shape=jax.ShapeDtypeStruct((B, _K, N), jnp.int32),
    )(x)


def _make_sc_gather(B, d, N):
    KN = _K * N
    n_groups = N // 16
    mesh = plsc.VectorSubcoreMesh(core_axis_name="c", subcore_axis_name="s")

    @functools.partial(
        pl.kernel,
        mesh=mesh,
        compiler_params=pltpu.CompilerParams(needs_layout_passes=False),
        out_type=jax.ShapeDtypeStruct((B * 2 * d * KN,), jnp.float32),
        scratch_types=[
            pltpu.VMEM((KN,), jnp.int32),
            pltpu.VMEM((N,), jnp.float32),
            pltpu.VMEM((KN,), jnp.float32),
            pltpu.VMEM((KN,), jnp.float32),
            pltpu.SemaphoreType.DMA,
            pltpu.SemaphoreType.DMA,
        ],
    )
    def sc_gather(xf_hbm, idxf_hbm, outf_hbm, idxbuf, xrow, bufa, bufb,
                  sema, semb):
        wid = lax.axis_index("s") * 2 + lax.axis_index("c")
        b = wid // 4
        cg = wid % 4
        pltpu.sync_copy(idxf_hbm.at[pl.ds(b * KN, KN)], idxbuf)

        bufs = (bufa, bufb)
        sems = (sema, semb)

        def fill_feature(buf):
            def body(tg, carry):
                iv = idxbuf[pl.ds(tg * 16, 16)]
                fv = plsc.load_gather(xrow, [iv])
                cv = xrow[pl.ds((tg % n_groups) * 16, 16)]
                buf[pl.ds(tg * 16, 16)] = fv - cv
                return carry
            lax.fori_loop(0, _K * n_groups, body, 0, unroll=8)

        def fill_center(buf):
            def body(tg, carry):
                buf[pl.ds(tg * 16, 16)] = xrow[pl.ds((tg % n_groups) * 16, 16)]
                return carry
            lax.fori_loop(0, _K * n_groups, body, 0, unroll=8)

        jobs = []
        for ci in range(16):
            jobs.append((True, ci))
            jobs.append((False, ci))
        copies = [None, None]
        for j, (is_feat, ci) in enumerate(jobs):
            slot = j % 2
            if is_feat:
                c = 16 * cg + ci
                pltpu.sync_copy(xf_hbm.at[pl.ds((b * d + c) * N, N)], xrow)
            if copies[slot] is not None:
                copies[slot].wait()
            buf = bufs[slot]
            if is_feat:
                fill_feature(buf)
                off = ((b * 2 * d) + 16 * cg + ci) * KN
            else:
                fill_center(buf)
                off = ((b * 2 * d) + d + 16 * cg + ci) * KN
            cp = pltpu.make_async_copy(buf, outf_hbm.at[pl.ds(off, KN)],
                                       sems[slot])
            cp.start()
            copies[slot] = cp
        copies[0].wait()
        copies[1].wait()

    return sc_gather


def kernel(x):
    B, d, N = x.shape
    idx = _topk(x)
    xf = x.reshape(-1)
    idxf = idx.reshape(-1)
    outf = _make_sc_gather(B, d, N)(xf, idxf)
    out = outf.reshape(B, 2 * d, _K, N)
    return jnp.transpose(out, (0, 1, 3, 2))


# speedup vs baseline: 1.4623x; 1.4125x over previous
import functools

import jax
import jax.numpy as jnp
from jax import lax
from jax.experimental import pallas as pl
from jax.experimental.pallas import tpu as pltpu
from jax.experimental.pallas import tpu_sc as plsc

_K = 20
_RB = 256


def _topk_kernel(x_ref, idx_ref, cent_ref):
    nb = pl.program_id(1)
    n0 = nb * _RB
    _, d, N = x_ref.shape

    x8 = x_ref[0, 0:8, :]
    xr8 = x_ref[0, 0:8, pl.ds(n0, _RB)]
    row = lax.broadcasted_iota(jnp.int32, (8, N), 0)
    rowr = lax.broadcasted_iota(jnp.int32, (8, _RB), 0)

    x3 = jnp.where(row < 3, x8, 0.0)
    xr3 = jnp.where(rowr < 3, 2.0 * xr8, 0.0)

    inner2 = lax.dot_general(x3, xr3, (((0,), (0,)), ((), ())),
                             preferred_element_type=jnp.float32)
    ones_col = jnp.ones((8, 1), jnp.float32)
    xxcol = lax.dot_general(x3 * x3, ones_col, (((0,), (0,)), ((), ())),
                            precision=lax.Precision.HIGHEST,
                            preferred_element_type=jnp.float32)
    p = inner2 - xxcol

    sub = lax.broadcasted_iota(jnp.int32, (N, _RB), 0)
    xr = x_ref[0, :, pl.ds(n0, _RB)]
    neg = jnp.float32(-jnp.inf)
    for t in range(_K):
        it = jnp.argmax(p, axis=0).astype(jnp.int32)[None, :]
        idx_ref[0, t, :] = it[0]
        cent_ref[0, :, t, :] = xr
        p = jnp.where(sub == it, neg, p)


def _topk(x):
    B, d, N = x.shape
    return pl.pallas_call(
        _topk_kernel,
        grid=(B, N // _RB),
        in_specs=[pl.BlockSpec((1, d, N), lambda b, nb: (b, 0, 0))],
        out_specs=[
            pl.BlockSpec((1, _K, _RB), lambda b, nb: (b, 0, nb)),
            pl.BlockSpec((1, d, _K, _RB), lambda b, nb: (b, 0, 0, nb)),
        ],
        out_shape=[
            jax.ShapeDtypeStruct((B, _K, N), jnp.int32),
            jax.ShapeDtypeStruct((B, d, _K, N), jnp.float32),
        ],
    )(x)


def _make_sc_gather(B, d, N):
    KN = _K * N
    mesh = plsc.VectorSubcoreMesh(core_axis_name="c", subcore_axis_name="s")

    @functools.partial(
        pl.kernel,
        mesh=mesh,
        compiler_params=pltpu.CompilerParams(needs_layout_passes=False),
        out_type=jax.ShapeDtypeStruct((B * d * KN,), jnp.float32),
        scratch_types=[
            pltpu.VMEM((KN,), jnp.int32),
            pltpu.VMEM((N,), jnp.float32),
            pltpu.VMEM((KN,), jnp.float32),
            pltpu.VMEM((KN,), jnp.float32),
            pltpu.SemaphoreType.DMA,
            pltpu.SemaphoreType.DMA,
        ],
    )
    def sc_gather(xf_hbm, idxf_hbm, outf_hbm, idxbuf, xrow, bufa, bufb,
                  sema, semb):
        wid = lax.axis_index("s") * 2 + lax.axis_index("c")
        b = wid // 4
        cg = wid % 4
        pltpu.sync_copy(idxf_hbm.at[pl.ds(b * KN, KN)], idxbuf)

        bufs = (bufa, bufb)
        sems = (sema, semb)

        def fill_feature(buf):
            def t_body(t, carry):
                base = t * N

                @plsc.parallel_loop(0, N, 16, unroll=8)
                def _(g):
                    iv = idxbuf[pl.ds(base + g, 16)]
                    fv = plsc.load_gather(xrow, [iv])
                    buf[pl.ds(base + g, 16)] = fv - xrow[pl.ds(g, 16)]

                return carry
            lax.fori_loop(0, _K, t_body, 0)

        copies = [None, None]
        for ci in range(16):
            slot = ci % 2
            c = 16 * cg + ci
            pltpu.sync_copy(xf_hbm.at[pl.ds((b * d + c) * N, N)], xrow)
            if copies[slot] is not None:
                copies[slot].wait()
            buf = bufs[slot]
            fill_feature(buf)
            off = ((b * d) + c) * KN
            cp = pltpu.make_async_copy(buf, outf_hbm.at[pl.ds(off, KN)],
                                       sems[slot])
            cp.start()
            copies[slot] = cp
        copies[0].wait()
        copies[1].wait()

    return sc_gather


def kernel(x):
    B, d, N = x.shape
    idx, cent = _topk(x)
    xf = x.reshape(-1)
    idxf = idx.reshape(-1)
    featf = _make_sc_gather(B, d, N)(xf, idxf)
    feat4 = featf.reshape(B, d, _K, N)
    out = jnp.concatenate([feat4, cent], axis=1)
    return jnp.transpose(out, (0, 1, 3, 2))


# speedup vs baseline: 1.4997x; 1.0256x over previous
import functools

import jax
import jax.numpy as jnp
from jax import lax
from jax.experimental import pallas as pl
from jax.experimental.pallas import tpu as pltpu
from jax.experimental.pallas import tpu_sc as plsc

_K = 20
_RB = 256


def _topk_kernel(x_ref, idx_ref, cent_ref):
    nb = pl.program_id(1)
    n0 = nb * _RB
    _, d, N = x_ref.shape

    x8 = x_ref[0, 0:8, :]
    xr8 = x_ref[0, 0:8, pl.ds(n0, _RB)]
    row = lax.broadcasted_iota(jnp.int32, (8, N), 0)
    rowr = lax.broadcasted_iota(jnp.int32, (8, _RB), 0)

    x3 = jnp.where(row < 3, x8, 0.0)
    xr3 = jnp.where(rowr < 3, 2.0 * xr8, 0.0)

    inner2 = lax.dot_general(x3, xr3, (((0,), (0,)), ((), ())),
                             preferred_element_type=jnp.float32)
    xxfull = jnp.sum(x3 * x3, axis=0, keepdims=True)
    xxcol = jnp.transpose(xxfull)
    x3r = jnp.where(rowr < 3, xr8, 0.0)
    xxrow = jnp.sum(x3r * x3r, axis=0, keepdims=True)
    p = (inner2 - xxcol) - xxrow

    sub = lax.broadcasted_iota(jnp.int32, (N, _RB), 0)
    xr = x_ref[0, :, pl.ds(n0, _RB)]
    neg = jnp.float32(-jnp.inf)
    for t in range(_K):
        it = jnp.argmax(p, axis=0).astype(jnp.int32)[None, :]
        idx_ref[0, t, :] = it[0]
        cent_ref[0, :, t, :] = xr
        p = jnp.where(sub == it, neg, p)


def _topk(x):
    B, d, N = x.shape
    return pl.pallas_call(
        _topk_kernel,
        grid=(B, N // _RB),
        in_specs=[pl.BlockSpec((1, d, N), lambda b, nb: (b, 0, 0))],
        out_specs=[
            pl.BlockSpec((1, _K, _RB), lambda b, nb: (b, 0, nb)),
            pl.BlockSpec((1, d, _K, _RB), lambda b, nb: (b, 0, 0, nb)),
        ],
        out_shape=[
            jax.ShapeDtypeStruct((B, _K, N), jnp.int32),
            jax.ShapeDtypeStruct((B, d, _K, N), jnp.float32),
        ],
    )(x)


def _make_sc_gather(B, d, N):
    KN = _K * N
    mesh = plsc.VectorSubcoreMesh(core_axis_name="c", subcore_axis_name="s")

    @functools.partial(
        pl.kernel,
        mesh=mesh,
        compiler_params=pltpu.CompilerParams(needs_layout_passes=False),
        out_type=jax.ShapeDtypeStruct((B * d * KN,), jnp.float32),
        scratch_types=[
            pltpu.VMEM((KN,), jnp.int32),
            pltpu.VMEM((N,), jnp.float32),
            pltpu.VMEM((KN,), jnp.float32),
            pltpu.VMEM((KN,), jnp.float32),
            pltpu.SemaphoreType.DMA,
            pltpu.SemaphoreType.DMA,
        ],
    )
    def sc_gather(xf_hbm, idxf_hbm, outf_hbm, idxbuf, xrow, bufa, bufb,
                  sema, semb):
        wid = lax.axis_index("s") * 2 + lax.axis_index("c")
        b = wid // 4
        cg = wid % 4
        pltpu.sync_copy(idxf_hbm.at[pl.ds(b * KN, KN)], idxbuf)

        bufs = (bufa, bufb)
        sems = (sema, semb)

        def fill_feature(buf):
            def t_body(t, carry):
                base = t * N

                @plsc.parallel_loop(0, N, 16, unroll=8)
                def _(g):
                    iv = idxbuf[pl.ds(base + g, 16)]
                    fv = plsc.load_gather(xrow, [iv])
                    buf[pl.ds(base + g, 16)] = fv - xrow[pl.ds(g, 16)]

                return carry
            lax.fori_loop(0, _K, t_body, 0)

        copies = [None, None]
        for ci in range(16):
            slot = ci % 2
            c = 16 * cg + ci
            pltpu.sync_copy(xf_hbm.at[pl.ds((b * d + c) * N, N)], xrow)
            if copies[slot] is not None:
                copies[slot].wait()
            buf = bufs[slot]
            fill_feature(buf)
            off = ((b * d) + c) * KN
            cp = pltpu.make_async_copy(buf, outf_hbm.at[pl.ds(off, KN)],
                                       sems[slot])
            cp.start()
            copies[slot] = cp
        copies[0].wait()
        copies[1].wait()

    return sc_gather


def kernel(x):
    B, d, N = x.shape
    idx, cent = _topk(x)
    xf = x.reshape(-1)
    idxf = idx.reshape(-1)
    featf = _make_sc_gather(B, d, N)(xf, idxf)
    feat4 = featf.reshape(B, d, _K, N)
    out = jnp.concatenate([feat4, cent], axis=1)
    return jnp.transpose(out, (0, 1, 3, 2))
